# baseline (speedup 1.0000x reference)
import functools

import jax
import jax.numpy as jnp
from jax import lax
from jax.experimental import pallas as pl
from jax.experimental.pallas import tpu as pltpu
from jax.experimental.pallas import tpu_sc as plsc

MB = 13
NRING = 3


def _band_blocks_kernel(emb_ref, out_ref, *, K, D, NT):
    step = pl.program_id(0)
    for mm in range(MB):
        m = step * MB + mm
        kk = jax.lax.broadcasted_iota(jnp.int32, (K, 512), 0)
        x = jax.lax.broadcasted_iota(jnp.int32, (K, 512), 1)
        pos = jnp.clip(x - 128 - m, 0, K - 1)
        pos = jnp.where(m == NT - 2, 0, pos)
        pos = jnp.where(m == NT - 1, K - 1, pos)
        oh = (kk == pos).astype(jnp.float32)
        res = jax.lax.dot_general(
            emb_ref[...], oh, (((0,), (0,)), ((), ())),
            preferred_element_type=jnp.float32)
        out_ref[pl.ds(D * mm, D), :] = res


def _build_sc_kernel(sq, sv, D, n_shift):
    info = plsc.get_sparse_core_info()
    NC = info.num_cores
    NW = NC * info.num_subcores
    P = sq // NW
    ntiles = sv // 128
    LO = NRING * 512
    HI = LO + 128
    SLANES = HI + 128

    mesh = plsc.VectorSubcoreMesh(core_axis_name="c", subcore_axis_name="s")

    @functools.partial(
        pl.kernel, mesh=mesh,
        out_type=jax.ShapeDtypeStruct((sq * D, sv), jnp.float32),
        scratch_types=[
            pltpu.VMEM((D, SLANES), jnp.float32),
            pltpu.SemaphoreType.DMA,
            pltpu.SemaphoreType.DMA,
            pltpu.SemaphoreType.DMA,
        ],
    )
    def k(b_hbm, out_hbm, s_ref, sem_b, sem_bs, sem_s):
        wid = lax.axis_index("s") * NC + lax.axis_index("c")
        base = wid * P

        def load_band(plane, slot):
            m = lax.rem(base + plane + 64, 128)
            pltpu.async_copy(
                b_hbm.at[pl.ds(pl.multiple_of(D * m, 8), D), :],
                s_ref.at[:, pl.ds(pl.multiple_of(512 * slot, 128), 512)],
                sem_b)

        def dummy_wait(sem, lanes):
            pltpu.make_async_copy(
                out_hbm.at[pl.ds(0, D), pl.ds(0, lanes)],
                s_ref.at[:, pl.ds(0, lanes)], sem).wait()

        pltpu.sync_copy(b_hbm.at[pl.ds(D * n_shift, D), pl.ds(0, 128)],
                        s_ref.at[:, pl.ds(LO, 128)])
        pltpu.sync_copy(b_hbm.at[pl.ds(D * (n_shift + 1), D), pl.ds(0, 128)],
                        s_ref.at[:, pl.ds(HI, 128)])

        for q in range(NRING - 1):
            load_band(q, q)

        def plane_body(p, _):
            i = base + p
            t0 = lax.div(i + 64, 128) - 1
            bt = jnp.clip(t0, 0, ntiles - 2)
            row0 = pl.multiple_of(D * i, 8)
            slot = lax.rem(p, NRING)

            @pl.when(p >= 1)
            def _():
                dummy_wait(sem_bs, 256)

            @pl.when(p + NRING - 1 < P)
            def _():
                load_band(p + NRING - 1, lax.rem(p + NRING - 1, NRING))

            dummy_wait(sem_b, 512)

            soff = pl.multiple_of(512 * slot + 128 * (1 + bt - t0), 128)
            pltpu.async_copy(
                s_ref.at[:, pl.ds(soff, 256)],
                out_hbm.at[pl.ds(row0, D),
                           pl.ds(pl.multiple_of(128 * bt, 128), 256)],
                sem_bs)

            for t in range(ntiles):
                @pl.when(jnp.logical_and(t != bt, t != bt + 1))
                def _():
                    src = pl.multiple_of(
                        jnp.where(t < bt, LO, HI), 128)
                    pltpu.async_copy(
                        s_ref.at[:, pl.ds(src, 128)],
                        out_hbm.at[pl.ds(row0, D), pl.ds(128 * t, 128)],
                        sem_s)

            return 0

        lax.fori_loop(0, P, plane_body, 0)

        dummy_wait(sem_bs, 256)

        def drain_body(p, _):
            dummy_wait(sem_s, SLANES)
            return 0

        lax.fori_loop(0, P, drain_body, 0)

    return k


def kernel(q, v, embeddings):
    sq, sv = q.shape[1], v.shape[1]
    K, D = embeddings.shape
    n_shift = 128
    nt = n_shift + 2

    b_all = pl.pallas_call(
        functools.partial(_band_blocks_kernel, K=K, D=D, NT=nt),
        grid=(nt // MB,),
        in_specs=[pl.BlockSpec((K, D), lambda s: (0, 0))],
        out_specs=pl.BlockSpec((MB * D, 512), lambda s: (s, 0)),
        out_shape=jax.ShapeDtypeStruct((nt * D, 512), jnp.float32),
    )(embeddings)

    sck = _build_sc_kernel(sq, sv, D, n_shift)
    out2 = sck(b_all)
    return out2.reshape(sq, D, sv).transpose(0, 2, 1)


# speedup vs baseline: 1.1245x; 1.1245x over previous
import functools

import jax
import jax.numpy as jnp
from jax import lax
from jax.experimental import pallas as pl
from jax.experimental.pallas import tpu as pltpu
from jax.experimental.pallas import tpu_sc as plsc

MB = 13
NRING = 3


def _band_blocks_kernel(emb_ref, out_ref, *, K, D, NT):
    step = pl.program_id(0)
    for mm in range(MB):
        m = step * MB + mm
        kk = jax.lax.broadcasted_iota(jnp.int32, (K, 256), 0)
        x = jax.lax.broadcasted_iota(jnp.int32, (K, 256), 1)
        pos = jnp.clip(x - m, 0, K - 1)
        pos = jnp.where(m == NT - 2, 0, pos)
        pos = jnp.where(m == NT - 1, K - 1, pos)
        oh = (kk == pos).astype(jnp.float32)
        res = jax.lax.dot_general(
            emb_ref[...], oh, (((0,), (0,)), ((), ())),
            preferred_element_type=jnp.float32)
        out_ref[pl.ds(D * mm, D), :] = res


def _build_sc_kernel(sq, sv, D, n_shift):
    info = plsc.get_sparse_core_info()
    NC = info.num_cores
    NW = NC * info.num_subcores
    P = sq // NW
    ntiles = sv // 128
    LO = 0
    HI = 384
    SLANES = NRING * 512

    mesh = plsc.VectorSubcoreMesh(core_axis_name="c", subcore_axis_name="s")

    @functools.partial(
        pl.kernel, mesh=mesh,
        out_type=jax.ShapeDtypeStruct((sq * D, sv), jnp.float32),
        scratch_types=[
            pltpu.VMEM((D, SLANES), jnp.float32),
            pltpu.SemaphoreType.DMA,
            pltpu.SemaphoreType.DMA,
            pltpu.SemaphoreType.DMA,
        ],
    )
    def k(b_hbm, out_hbm, s_ref, sem_b, sem_bs, sem_s):
        wid = lax.axis_index("s") * NC + lax.axis_index("c")
        base = wid * P

        def load_band(plane, slot):
            m = lax.rem(base + plane + 64, 128)
            pltpu.async_copy(
                b_hbm.at[pl.ds(pl.multiple_of(D * m, 8), D), :],
                s_ref.at[:, pl.ds(pl.multiple_of(512 * slot + 128, 128), 256)],
                sem_b)

        def dummy_wait(sem, lanes):
            pltpu.make_async_copy(
                out_hbm.at[pl.ds(0, D), pl.ds(0, lanes)],
                s_ref.at[:, pl.ds(0, lanes)], sem).wait()

        for f in range(NRING):
            pltpu.sync_copy(b_hbm.at[pl.ds(D * n_shift, D), pl.ds(0, 128)],
                            s_ref.at[:, pl.ds(512 * f, 128)])
            pltpu.sync_copy(
                b_hbm.at[pl.ds(D * (n_shift + 1), D), pl.ds(0, 128)],
                s_ref.at[:, pl.ds(512 * f + 384, 128)])

        for q in range(NRING - 1):
            load_band(q, q)

        def plane_body(p, _):
            i = base + p
            t0 = lax.div(i + 64, 128) - 1
            bt = jnp.clip(t0, 0, ntiles - 2)
            row0 = pl.multiple_of(D * i, 8)
            slot = lax.rem(p, NRING)

            @pl.when(p >= 1)
            def _():
                dummy_wait(sem_bs, 256)

            @pl.when(p + NRING - 1 < P)
            def _():
                load_band(p + NRING - 1, lax.rem(p + NRING - 1, NRING))

            dummy_wait(sem_b, 256)

            soff = pl.multiple_of(512 * slot + 128 * (1 + bt - t0), 128)
            pltpu.async_copy(
                s_ref.at[:, pl.ds(soff, 256)],
                out_hbm.at[pl.ds(row0, D),
                           pl.ds(pl.multiple_of(128 * bt, 128), 256)],
                sem_bs)

            for t in range(ntiles):
                @pl.when(jnp.logical_and(t != bt, t != bt + 1))
                def _():
                    src = pl.multiple_of(
                        jnp.where(t < bt, LO, HI), 128)
                    pltpu.async_copy(
                        s_ref.at[:, pl.ds(src, 128)],
                        out_hbm.at[pl.ds(row0, D), pl.ds(128 * t, 128)],
                        sem_s)

            return 0

        lax.fori_loop(0, P, plane_body, 0)

        dummy_wait(sem_bs, 256)

        def drain_body(p, _):
            dummy_wait(sem_s, 896)
            dummy_wait(sem_s, 896)
            return 0

        lax.fori_loop(0, P, drain_body, 0)

    return k


def kernel(q, v, embeddings):
    sq, sv = q.shape[1], v.shape[1]
    K, D = embeddings.shape
    n_shift = 128
    nt = n_shift + 2

    b_all = pl.pallas_call(
        functools.partial(_band_blocks_kernel, K=K, D=D, NT=nt),
        grid=(nt // MB,),
        in_specs=[pl.BlockSpec((K, D), lambda s: (0, 0))],
        out_specs=pl.BlockSpec((MB * D, 256), lambda s: (s, 0)),
        out_shape=jax.ShapeDtypeStruct((nt * D, 256), jnp.float32),
    )(embeddings)

    sck = _build_sc_kernel(sq, sv, D, n_shift)
    out2 = sck(b_all)
    return out2.reshape(sq, D, sv).transpose(0, 2, 1)
